# Initial kernel scaffold; baseline (speedup 1.0000x reference)
import jax
import jax.numpy as jnp
from jax.experimental import pallas as pl


def kernel(x, W):
    raise NotImplementedError("write your pallas kernel here")


# speedup vs baseline: 1.8789x; 1.8789x over previous
import functools

import jax
import jax.numpy as jnp
from jax import lax
from jax.experimental import pallas as pl
from jax.experimental.pallas import tpu as pltpu
from jax.experimental.pallas import tpu_sc as plsc

EMBED_DIM = 64
CHUNK = 128
NBUF = 4


def _make_kernel(total, num_workers):
    per_w = total // num_workers
    n_chunks = per_w // CHUNK
    n_groups = n_chunks // NBUF
    assert per_w * num_workers == total
    assert n_chunks * CHUNK == per_w
    assert n_groups * NBUF == n_chunks

    mesh = plsc.VectorSubcoreMesh(core_axis_name="c", subcore_axis_name="s")
    num_cores = mesh.num_cores

    @functools.partial(
        pl.kernel,
        out_type=jax.ShapeDtypeStruct((total, EMBED_DIM), jnp.float32),
        mesh=mesh,
        scratch_types=[
            pltpu.VMEM((n_chunks, CHUNK), jnp.int32),
            pltpu.VMEM((NBUF, CHUNK, EMBED_DIM), jnp.float32),
        ] + [pltpu.SemaphoreType.DMA] * NBUF,
        compiler_params=pltpu.CompilerParams(use_tc_tiling_on_sc=False),
    )
    def emb_kernel(x_hbm, w_hbm, out_hbm, idx_v, rows_v, *gsems):
        wid = lax.axis_index("s") * num_cores + lax.axis_index("c")
        row_base = wid * n_chunks
        out_base = wid * per_w

        pltpu.sync_copy(x_hbm.at[pl.ds(row_base, n_chunks)], idx_v)

        def start_gather(chunk, buf):
            pltpu.make_async_copy(
                w_hbm.at[idx_v.at[chunk]], rows_v.at[buf], gsems[buf]
            ).start()

        def wait_gather(chunk, buf):
            pltpu.make_async_copy(
                w_hbm.at[idx_v.at[chunk]], rows_v.at[buf], gsems[buf]
            ).wait()

        for b in range(NBUF):
            start_gather(b, b)

        def group_body(g, _):
            for b in range(NBUF):
                i = g * NBUF + b
                wait_gather(i, b)
                pltpu.sync_copy(
                    rows_v.at[b],
                    out_hbm.at[pl.ds(out_base + i * CHUNK, CHUNK)],
                )
                nxt = i + NBUF

                @pl.when(nxt < n_chunks)
                def _():
                    start_gather(nxt, b)

        lax.fori_loop(0, n_groups, group_body, None)

    return emb_kernel


def kernel(x, W):
    batch, hist = x.shape
    total = batch * hist
    info = plsc.get_sparse_core_info()
    num_workers = info.num_cores * info.num_subcores
    x_rows = x.reshape(total // CHUNK, CHUNK).astype(jnp.int32)
    out = _make_kernel(total, num_workers)(x_rows, W)
    return out.reshape(batch, hist, EMBED_DIM)
